# baseline (speedup 1.0000x reference)
import functools

import jax
import jax.numpy as jnp
from jax import lax
from jax.experimental import pallas as pl
from jax.experimental.pallas import tpu as pltpu
from jax.experimental.pallas import tpu_sc as plsc

D_MODEL = 768
N_EXP = 16
T_TOK = 4096
D_FF = 3072
BT = 512
MAX_TILES = 24
PAD_T = MAX_TILES * BT
N_WORKERS = 32
CHUNK = T_TOK // N_WORKERS


def _gelu_exact(h):
    return 0.5 * h * (1.0 + lax.erf(h * 0.7071067811865476))


_RB = 8
_RBT = T_TOK // _RB


def _route_body(x_ref, gw_ref, pos_ref, te_ref, tv_ref, sc_ref):
    i = pl.program_id(0)
    sc_ref[pl.ds(i * _RBT, _RBT), :] = jnp.dot(
        x_ref[...], gw_ref[...], preferred_element_type=jnp.float32)

    @pl.when(i == _RB - 1)
    def _():
        _route_tail(sc_ref[...], pos_ref, te_ref, tv_ref)


def _route_tail(scores, pos_ref, te_ref, tv_ref):
    lane = lax.broadcasted_iota(jnp.int32, (T_TOK, N_EXP), 1)
    m = jnp.max(scores, axis=1, keepdims=True)
    cand = jnp.where(scores >= m, lane, jnp.int32(1 << 20))
    eidx = jnp.min(cand, axis=1, keepdims=True)
    onehot = (lane == eidx).astype(jnp.int32)

    nch, ch = T_TOK // 128, 128
    ohf = onehot.astype(jnp.float32).reshape(nch, ch, N_EXP)
    li = lax.broadcasted_iota(jnp.int32, (ch, ch), 0)
    lj = lax.broadcasted_iota(jnp.int32, (ch, ch), 1)
    lt_incl = (lj <= li).astype(jnp.float32)
    incl_c = lax.dot_general(
        jnp.broadcast_to(lt_incl, (nch, ch, ch)), ohf,
        (((2,), (1,)), ((0,), (0,))),
        preferred_element_type=jnp.float32)
    chunk_tot = jnp.sum(ohf, axis=1)
    ci = lax.broadcasted_iota(jnp.int32, (nch, nch), 0)
    cj = lax.broadcasted_iota(jnp.int32, (nch, nch), 1)
    lt_strict = (cj < ci).astype(jnp.float32)
    off = jnp.dot(lt_strict, chunk_tot,
                  preferred_element_type=jnp.float32)
    incl = (incl_c + off[:, None, :]).reshape(T_TOK, N_EXP).astype(jnp.int32)
    rank = jnp.sum(onehot * incl, axis=1, keepdims=True) - 1

    counts = jnp.sum(onehot, axis=0, keepdims=True)
    tiles_e = (counts + BT - 1) // BT
    padded = tiles_e * BT
    lti = lax.broadcasted_iota(jnp.int32, (N_EXP, N_EXP), 0)
    ltj = lax.broadcasted_iota(jnp.int32, (N_EXP, N_EXP), 1)
    ltmask = (lti < ltj).astype(jnp.float32)
    pad_start = jnp.dot(padded.astype(jnp.float32), ltmask,
                        preferred_element_type=jnp.float32).astype(jnp.int32)
    tile_end = (pad_start + padded) // BT
    total_tiles = jnp.sum(tiles_e)

    pos = jnp.sum(onehot * pad_start, axis=1, keepdims=True) + rank
    pos_ref[...] = pos

    jrow = lax.broadcasted_iota(jnp.int32, (MAX_TILES, N_EXP), 0)
    expert_raw = jnp.sum((jnp.broadcast_to(tile_end, (MAX_TILES, N_EXP)) <= jrow)
                         .astype(jnp.int32), axis=1, keepdims=True)
    valid = (jrow[:, 0:1] < total_tiles)
    elane = lax.broadcasted_iota(jnp.int32, (1, N_EXP), 1)
    last_e = jnp.max(jnp.where(tiles_e > 0, elane, -1))
    te_ref[...] = jnp.where(valid, expert_raw, last_e)
    tv_ref[...] = valid.astype(jnp.int32)


def _route(xf, gate_w):
    return pl.pallas_call(
        _route_body,
        grid=(_RB,),
        in_specs=[
            pl.BlockSpec((_RBT, D_MODEL), lambda i: (i, 0)),
            pl.BlockSpec((D_MODEL, N_EXP), lambda i: (0, 0)),
        ],
        out_specs=(
            pl.BlockSpec((T_TOK, 1), lambda i: (0, 0)),
            pl.BlockSpec((MAX_TILES, 1), lambda i: (0, 0)),
            pl.BlockSpec((MAX_TILES, 1), lambda i: (0, 0)),
        ),
        out_shape=(
            jax.ShapeDtypeStruct((T_TOK, 1), jnp.int32),
            jax.ShapeDtypeStruct((MAX_TILES, 1), jnp.int32),
            jax.ShapeDtypeStruct((MAX_TILES, 1), jnp.int32),
        ),
        scratch_shapes=[pltpu.VMEM((T_TOK, N_EXP), jnp.float32)],
        compiler_params=pltpu.CompilerParams(
            dimension_semantics=("arbitrary",)),
    )(xf, gate_w)


def _sc_scatter(xf, pos):
    mesh = plsc.VectorSubcoreMesh(core_axis_name="c", subcore_axis_name="s")

    half = CHUNK // 2

    @functools.partial(
        pl.kernel, mesh=mesh,
        out_type=jax.ShapeDtypeStruct((PAD_T, D_MODEL), jnp.float32),
        scratch_types=[
            pltpu.VMEM((half,), jnp.int32),
            pltpu.VMEM((half,), jnp.int32),
            pltpu.VMEM((half, D_MODEL), jnp.float32),
            pltpu.VMEM((half, D_MODEL), jnp.float32),
            pltpu.SemaphoreType.DMA,
            pltpu.SemaphoreType.DMA,
            pltpu.SemaphoreType.DMA,
            pltpu.SemaphoreType.DMA,
        ],
    )
    def k(x_hbm, pos_hbm, xpad_hbm, idx0, idx1, rows0, rows1,
          s0, s1, s2, s3):
        wid = lax.axis_index("s") * 2 + lax.axis_index("c")
        base = wid * CHUNK
        c0 = pltpu.async_copy(x_hbm.at[pl.ds(base, half)], rows0, s0)
        c1 = pltpu.async_copy(x_hbm.at[pl.ds(base + half, half)], rows1, s1)
        pltpu.sync_copy(pos_hbm.at[pl.ds(base, half)], idx0)
        pltpu.sync_copy(pos_hbm.at[pl.ds(base + half, half)], idx1)
        c0.wait()
        w0 = pltpu.async_copy(rows0, xpad_hbm.at[idx0], s2)
        c1.wait()
        w1 = pltpu.async_copy(rows1, xpad_hbm.at[idx1], s3)
        w0.wait()
        w1.wait()

    return k(xf, pos)


def _sc_gather(y_pad, pos):
    mesh = plsc.VectorSubcoreMesh(core_axis_name="c", subcore_axis_name="s")

    half = CHUNK // 2

    @functools.partial(
        pl.kernel, mesh=mesh,
        out_type=jax.ShapeDtypeStruct((T_TOK, D_MODEL), jnp.float32),
        scratch_types=[
            pltpu.VMEM((half,), jnp.int32),
            pltpu.VMEM((half,), jnp.int32),
            pltpu.VMEM((half, D_MODEL), jnp.float32),
            pltpu.VMEM((half, D_MODEL), jnp.float32),
            pltpu.SemaphoreType.DMA,
            pltpu.SemaphoreType.DMA,
            pltpu.SemaphoreType.DMA,
            pltpu.SemaphoreType.DMA,
        ],
    )
    def k(ypad_hbm, pos_hbm, out_hbm, idx0, idx1, rows0, rows1,
          s0, s1, s2, s3):
        wid = lax.axis_index("s") * 2 + lax.axis_index("c")
        base = wid * CHUNK
        pltpu.sync_copy(pos_hbm.at[pl.ds(base, half)], idx0)
        g0 = pltpu.async_copy(ypad_hbm.at[idx0], rows0, s0)
        pltpu.sync_copy(pos_hbm.at[pl.ds(base + half, half)], idx1)
        g1 = pltpu.async_copy(ypad_hbm.at[idx1], rows1, s1)
        g0.wait()
        w0 = pltpu.async_copy(rows0, out_hbm.at[pl.ds(base, half)], s2)
        g1.wait()
        w1 = pltpu.async_copy(rows1, out_hbm.at[pl.ds(base + half, half)], s3)
        w0.wait()
        w1.wait()

    return k(y_pad, pos)


def _mm_body(te_ref, tv_ref, x_ref, wfc_ref, wproj_ref, out_ref):
    j = pl.program_id(0)
    valid = tv_ref[j] == 1

    @pl.when(valid)
    def _():
        h = jnp.dot(x_ref[...], wfc_ref[0], preferred_element_type=jnp.float32)
        h = _gelu_exact(h)
        y = jnp.dot(h, wproj_ref[0], preferred_element_type=jnp.float32)
        out_ref[...] = y.astype(jnp.bfloat16).astype(jnp.float32)


def _grouped_mm(te, tv, x_pad, w_fc, w_proj):
    grid_spec = pltpu.PrefetchScalarGridSpec(
        num_scalar_prefetch=2,
        grid=(MAX_TILES,),
        in_specs=[
            pl.BlockSpec((BT, D_MODEL),
                         lambda j, te, tv: (jnp.where(tv[j] == 1, j, 0), 0)),
            pl.BlockSpec((1, D_MODEL, D_FF), lambda j, te, tv: (te[j], 0, 0)),
            pl.BlockSpec((1, D_FF, D_MODEL), lambda j, te, tv: (te[j], 0, 0)),
        ],
        out_specs=pl.BlockSpec(
            (BT, D_MODEL),
            lambda j, te, tv: (jnp.where(tv[j] == 1, j, MAX_TILES - 1), 0)),
    )
    return pl.pallas_call(
        _mm_body,
        grid_spec=grid_spec,
        out_shape=jax.ShapeDtypeStruct((PAD_T, D_MODEL), jnp.float32),
        compiler_params=pltpu.CompilerParams(
            dimension_semantics=("arbitrary",)),
    )(te, tv, x_pad, w_fc, w_proj)


def kernel(x, gate_w, w_fc, w_proj):
    orig_shape = x.shape
    xf = x.reshape(T_TOK, D_MODEL)
    pos2d, te2d, tv2d = _route(xf, gate_w)
    pos = pos2d.reshape(T_TOK)
    te = te2d.reshape(MAX_TILES)
    tv = tv2d.reshape(MAX_TILES)
    x_pad = _sc_scatter(xf, pos)
    y_pad = _grouped_mm(te, tv, x_pad, w_fc, w_proj)
    out = _sc_gather(y_pad, pos)
    return out.reshape(orig_shape)


# speedup vs baseline: 1.0288x; 1.0288x over previous
import functools

import jax
import jax.numpy as jnp
from jax import lax
from jax.experimental import pallas as pl
from jax.experimental.pallas import tpu as pltpu
from jax.experimental.pallas import tpu_sc as plsc

D_MODEL = 768
N_EXP = 16
T_TOK = 4096
D_FF = 3072
BT = 512
MAX_TILES = 24
PAD_T = MAX_TILES * BT
N_WORKERS = 32
CHUNK = T_TOK // N_WORKERS


def _gelu_exact(h):
    return 0.5 * h * (1.0 + lax.erf(h * 0.7071067811865476))


def _route_body(x_ref, gw_ref, pos_ref, te_ref, tv_ref):
    scores = jnp.dot(x_ref[...], gw_ref[...],
                     preferred_element_type=jnp.float32)
    lane = lax.broadcasted_iota(jnp.int32, (T_TOK, N_EXP), 1)
    m = jnp.max(scores, axis=1, keepdims=True)
    cand = jnp.where(scores >= m, lane, jnp.int32(1 << 20))
    eidx = jnp.min(cand, axis=1, keepdims=True)
    onehot = (lane == eidx).astype(jnp.int32)

    nch, ch = T_TOK // 128, 128
    ohf = onehot.astype(jnp.float32).reshape(nch, ch, N_EXP)
    li = lax.broadcasted_iota(jnp.int32, (ch, ch), 0)
    lj = lax.broadcasted_iota(jnp.int32, (ch, ch), 1)
    lt_incl = (lj <= li).astype(jnp.float32)
    incl_c = lax.dot_general(
        jnp.broadcast_to(lt_incl, (nch, ch, ch)), ohf,
        (((2,), (1,)), ((0,), (0,))),
        preferred_element_type=jnp.float32)
    chunk_tot = jnp.sum(ohf, axis=1)
    ci = lax.broadcasted_iota(jnp.int32, (nch, nch), 0)
    cj = lax.broadcasted_iota(jnp.int32, (nch, nch), 1)
    lt_strict = (cj < ci).astype(jnp.float32)
    off = jnp.dot(lt_strict, chunk_tot,
                  preferred_element_type=jnp.float32)
    incl = (incl_c + off[:, None, :]).reshape(T_TOK, N_EXP).astype(jnp.int32)
    rank = jnp.sum(onehot * incl, axis=1, keepdims=True) - 1

    counts = jnp.sum(onehot, axis=0, keepdims=True)
    tiles_e = (counts + BT - 1) // BT
    padded = tiles_e * BT
    lti = lax.broadcasted_iota(jnp.int32, (N_EXP, N_EXP), 0)
    ltj = lax.broadcasted_iota(jnp.int32, (N_EXP, N_EXP), 1)
    ltmask = (lti < ltj).astype(jnp.float32)
    pad_start = jnp.dot(padded.astype(jnp.float32), ltmask,
                        preferred_element_type=jnp.float32).astype(jnp.int32)
    tile_end = (pad_start + padded) // BT
    total_tiles = jnp.sum(tiles_e)

    pos = jnp.sum(onehot * pad_start, axis=1, keepdims=True) + rank
    pos_ref[...] = pos

    jrow = lax.broadcasted_iota(jnp.int32, (MAX_TILES, N_EXP), 0)
    expert_raw = jnp.sum((jnp.broadcast_to(tile_end, (MAX_TILES, N_EXP)) <= jrow)
                         .astype(jnp.int32), axis=1, keepdims=True)
    valid = (jrow[:, 0:1] < total_tiles)
    elane = lax.broadcasted_iota(jnp.int32, (1, N_EXP), 1)
    last_e = jnp.max(jnp.where(tiles_e > 0, elane, -1))
    te_ref[...] = jnp.where(valid, expert_raw, last_e)
    tv_ref[...] = valid.astype(jnp.int32)


def _route(xf, gate_w):
    return pl.pallas_call(
        _route_body,
        out_shape=(
            jax.ShapeDtypeStruct((T_TOK, 1), jnp.int32),
            jax.ShapeDtypeStruct((MAX_TILES, 1), jnp.int32),
            jax.ShapeDtypeStruct((MAX_TILES, 1), jnp.int32),
        ),
    )(xf, gate_w)


def _sc_scatter(xf, pos):
    mesh = plsc.VectorSubcoreMesh(core_axis_name="c", subcore_axis_name="s")

    half = CHUNK // 2

    @functools.partial(
        pl.kernel, mesh=mesh,
        out_type=jax.ShapeDtypeStruct((PAD_T, D_MODEL), jnp.float32),
        scratch_types=[
            pltpu.VMEM((half,), jnp.int32),
            pltpu.VMEM((half,), jnp.int32),
            pltpu.VMEM((half, D_MODEL), jnp.float32),
            pltpu.VMEM((half, D_MODEL), jnp.float32),
            pltpu.SemaphoreType.DMA,
            pltpu.SemaphoreType.DMA,
            pltpu.SemaphoreType.DMA,
            pltpu.SemaphoreType.DMA,
        ],
    )
    def k(x_hbm, pos_hbm, xpad_hbm, idx0, idx1, rows0, rows1,
          s0, s1, s2, s3):
        wid = lax.axis_index("s") * 2 + lax.axis_index("c")
        base = wid * CHUNK
        c0 = pltpu.async_copy(x_hbm.at[pl.ds(base, half)], rows0, s0)
        c1 = pltpu.async_copy(x_hbm.at[pl.ds(base + half, half)], rows1, s1)
        pltpu.sync_copy(pos_hbm.at[pl.ds(base, half)], idx0)
        pltpu.sync_copy(pos_hbm.at[pl.ds(base + half, half)], idx1)
        c0.wait()
        w0 = pltpu.async_copy(rows0, xpad_hbm.at[idx0], s2)
        c1.wait()
        w1 = pltpu.async_copy(rows1, xpad_hbm.at[idx1], s3)
        w0.wait()
        w1.wait()

    return k(xf, pos)


def _sc_gather(y_pad, pos):
    mesh = plsc.VectorSubcoreMesh(core_axis_name="c", subcore_axis_name="s")

    half = CHUNK // 2

    @functools.partial(
        pl.kernel, mesh=mesh,
        out_type=jax.ShapeDtypeStruct((T_TOK, D_MODEL), jnp.float32),
        scratch_types=[
            pltpu.VMEM((half,), jnp.int32),
            pltpu.VMEM((half,), jnp.int32),
            pltpu.VMEM((half, D_MODEL), jnp.float32),
            pltpu.VMEM((half, D_MODEL), jnp.float32),
            pltpu.SemaphoreType.DMA,
            pltpu.SemaphoreType.DMA,
            pltpu.SemaphoreType.DMA,
            pltpu.SemaphoreType.DMA,
        ],
    )
    def k(ypad_hbm, pos_hbm, out_hbm, idx0, idx1, rows0, rows1,
          s0, s1, s2, s3):
        wid = lax.axis_index("s") * 2 + lax.axis_index("c")
        base = wid * CHUNK
        pltpu.sync_copy(pos_hbm.at[pl.ds(base, half)], idx0)
        g0 = pltpu.async_copy(ypad_hbm.at[idx0], rows0, s0)
        pltpu.sync_copy(pos_hbm.at[pl.ds(base + half, half)], idx1)
        g1 = pltpu.async_copy(ypad_hbm.at[idx1], rows1, s1)
        g0.wait()
        w0 = pltpu.async_copy(rows0, out_hbm.at[pl.ds(base, half)], s2)
        g1.wait()
        w1 = pltpu.async_copy(rows1, out_hbm.at[pl.ds(base + half, half)], s3)
        w0.wait()
        w1.wait()

    return k(y_pad, pos)


def _mm_body(te_ref, tv_ref, x_ref, wfc_ref, wproj_ref, out_ref):
    j = pl.program_id(0)
    valid = tv_ref[j] == 1

    @pl.when(valid)
    def _():
        h = jnp.dot(x_ref[...], wfc_ref[0], preferred_element_type=jnp.float32)
        h = _gelu_exact(h)
        y = jnp.dot(h, wproj_ref[0], preferred_element_type=jnp.float32)
        out_ref[...] = y.astype(jnp.bfloat16).astype(jnp.float32)


def _grouped_mm(te, tv, x_pad, w_fc, w_proj):
    grid_spec = pltpu.PrefetchScalarGridSpec(
        num_scalar_prefetch=2,
        grid=(MAX_TILES,),
        in_specs=[
            pl.BlockSpec((BT, D_MODEL),
                         lambda j, te, tv: (jnp.where(tv[j] == 1, j, 0), 0)),
            pl.BlockSpec((1, D_MODEL, D_FF), lambda j, te, tv: (te[j], 0, 0)),
            pl.BlockSpec((1, D_FF, D_MODEL), lambda j, te, tv: (te[j], 0, 0)),
        ],
        out_specs=pl.BlockSpec(
            (BT, D_MODEL),
            lambda j, te, tv: (jnp.where(tv[j] == 1, j, MAX_TILES - 1), 0)),
    )
    return pl.pallas_call(
        _mm_body,
        grid_spec=grid_spec,
        out_shape=jax.ShapeDtypeStruct((PAD_T, D_MODEL), jnp.float32),
        compiler_params=pltpu.CompilerParams(
            dimension_semantics=("arbitrary",)),
    )(te, tv, x_pad, w_fc, w_proj)


def kernel(x, gate_w, w_fc, w_proj):
    orig_shape = x.shape
    xf = x.reshape(T_TOK, D_MODEL)
    pos2d, te2d, tv2d = _route(xf, gate_w)
    pos = pos2d.reshape(T_TOK)
    te = te2d.reshape(MAX_TILES)
    tv = tv2d.reshape(MAX_TILES)
    x_pad = _sc_scatter(xf, pos)
    y_pad = _grouped_mm(te, tv, x_pad, w_fc, w_proj)
    out = _sc_gather(y_pad, pos)
    return out.reshape(orig_shape)
